# baseline (speedup 1.0000x reference)
import functools

import jax
import jax.numpy as jnp
from jax import lax
from jax.experimental import pallas as pl
from jax.experimental.pallas import tpu as pltpu
from jax.experimental.pallas import tpu_sc as plsc

B = 16384
NUM_DENSE = 13
NF = 26
NG = 1
NF_G = NF // NG
VOCAB = 100000
EMB = 32
HIDDEN0 = 256

VCAP = 106496
TR_CHUNK = 8192

NC, NS = 2, 16
NW = NC * NS
SUBS = 4
CHUNK = SUBS * 128


def _tc_transpose(tables_t, fbase):
    q = TR_CHUNK // 8

    def body(*refs):
        out_ref = refs[-1]
        z = jnp.concatenate([r[0] for r in refs[:-1]], axis=0)
        zp = pltpu.bitcast(z.astype(jnp.bfloat16), jnp.float32)
        out_ref[...] = jnp.transpose(zp, (1, 0))

    last_blk = (VOCAB - 1) // q

    def in_spec(j):
        return pl.BlockSpec(
            (1, EMB, q),
            lambda f, c, j=j: (f + fbase, 0, jnp.minimum(8 * c + j, last_blk)))

    return pl.pallas_call(
        body,
        grid=(NF_G, VCAP // TR_CHUNK),
        in_specs=[in_spec(j) for j in range(8)],
        out_specs=pl.BlockSpec((q, 128),
                               lambda f, c: (f * (VCAP // TR_CHUNK) + c, 0)),
        out_shape=jax.ShapeDtypeStruct((NF_G * VCAP // 8, 128), jnp.float32),
    )(*([tables_t] * 8))


def _sc_gather(sparse2d, tables_flat, gname):
    rows = B * NF_G
    rw = rows // NW
    idx_rows = rw // 128
    nchunk = rw // CHUNK
    mesh = plsc.VectorSubcoreMesh(core_axis_name="c", subcore_axis_name="s")

    @functools.partial(
        pl.kernel,
        out_type=jax.ShapeDtypeStruct((rows, 16), jnp.float32),
        name=gname,
        mesh=mesh,
        compiler_params=pltpu.CompilerParams(use_tc_tiling_on_sc=False),
        scratch_types=[
            pltpu.VMEM((idx_rows, 128), jnp.int32),
            pltpu.VMEM((CHUNK, 16), jnp.float32),
            pltpu.SemaphoreType.DMA,
        ],
    )
    def k(sparse_hbm, tables_hbm, out_hbm, idx_v, rows_v, sem):
        wid = lax.axis_index("s") * NC + lax.axis_index("c")
        base = wid * rw

        pltpu.sync_copy(sparse_hbm.at[pl.ds(wid * idx_rows, idx_rows)], idx_v)

        def row_body(i, carry):
            for j in range(128 // 16):
                g0 = base + i * 128 + j * 16
                lanes = g0 + lax.iota(jnp.int32, 16)
                off = (lanes % NF_G) * VCAP
                sl = pl.ds(j * 16, 16)
                v = idx_v[i, sl]
                p = (((v >> 13) << 13) + ((v & 1023) << 3) + ((v >> 10) & 7))
                idx_v[i, sl] = off + p
            return carry

        lax.fori_loop(0, idx_rows, row_body, 0)

        def chunk_body(c, carry):
            handles = []
            for s in range(SUBS):
                h = pltpu.async_copy(
                    tables_hbm.at[idx_v.at[c * SUBS + s]],
                    rows_v.at[pl.ds(s * 128, 128)],
                    sem,
                )
                handles.append(h)
            for h in handles:
                h.wait()
            pltpu.sync_copy(rows_v, out_hbm.at[pl.ds(base + c * CHUNK, CHUNK)])
            return carry

        lax.fori_loop(0, nchunk, chunk_body, 0)

    return k(sparse2d, tables_flat)


def _mlp(embeds, dense_pad, w0s, w0d, b0, w1, b1, w2, b2, bs=1024):
    h0 = b0.shape[-1]
    h1 = b1.shape[-1]
    h2 = b2.shape[-1]
    ne = len(embeds)
    de = embeds[0].shape[-1]
    dd = dense_pad.shape[-1]

    def body(*refs):
        e_refs = refs[:ne]
        d_ref = refs[ne]
        w_refs = refs[ne + 1: ne + 1 + 2 * ne]
        w0d_ref, b0_ref, w1_ref, b1_ref, w2_ref, b2_ref, out_ref = \
            refs[ne + 1 + 2 * ne:]
        x = jnp.dot(d_ref[...], w0d_ref[...],
                    preferred_element_type=jnp.float32)
        for g, e_ref in enumerate(e_refs):
            u = pltpu.bitcast(e_ref[...], jnp.bfloat16)
            u3 = u.reshape(bs, 2, de)
            x = x + jnp.dot(u3[:, 0, :], w_refs[2 * g][...],
                            preferred_element_type=jnp.float32)
            x = x + jnp.dot(u3[:, 1, :], w_refs[2 * g + 1][...],
                            preferred_element_type=jnp.float32)
        x = jnp.maximum(x + b0_ref[...], 0.0)
        x = jnp.maximum(
            jnp.dot(x, w1_ref[...], preferred_element_type=jnp.float32)
            + b1_ref[...], 0.0)
        x = jnp.maximum(
            jnp.dot(x, w2_ref[...], preferred_element_type=jnp.float32)
            + b2_ref[...], 0.0)
        out_ref[...] = x

    def full(shape):
        return pl.BlockSpec(shape, lambda i: tuple(0 for _ in shape))

    return pl.pallas_call(
        body,
        grid=(B // bs,),
        in_specs=(
            [pl.BlockSpec((bs, de), lambda i: (i, 0)) for _ in range(ne)]
            + [pl.BlockSpec((bs, dd), lambda i: (i, 0))]
            + [full((de, h0)) for _ in range(2 * ne)]
            + [full((dd, h0)), full((1, h0)), full((h0, h1)), full((1, h1)),
               full((h1, h2)), full((1, h2))]
        ),
        out_specs=pl.BlockSpec((bs, h2), lambda i: (i, 0)),
        out_shape=jax.ShapeDtypeStruct((B, h2), jnp.float32),
    )(*embeds, dense_pad, *w0s, w0d, b0.reshape(1, h0), w1,
      b1.reshape(1, h1), w2, b2.reshape(1, h2))


def kernel(dense_inputs, sparse_inputs, tables, W0, b0, W1, b1, W2, b2):
    sparse_i = sparse_inputs.astype(jnp.int32)
    tables_t = jnp.transpose(tables, (0, 2, 1))

    embeds = []
    for g in range(NG):
        sp = sparse_i[:, g * NF_G:(g + 1) * NF_G].reshape(-1, 128)
        packed = _tc_transpose(tables_t, g * NF_G)
        raw = _sc_gather(sp, packed.reshape(NF_G * VCAP, 16),
                         f"sc_embed_gather{g}")
        embeds.append(raw.reshape(B, NF_G * 16))

    dense_pad = jnp.pad(dense_inputs, ((0, 0), (0, 128 - NUM_DENSE)))
    w0d = jnp.pad(W0[:NUM_DENSE], ((0, 128 - NUM_DENSE), (0, 0)))
    w0s = []
    for g in range(NG):
        w0g = W0[NUM_DENSE + g * NF_G * EMB: NUM_DENSE + (g + 1) * NF_G * EMB]
        w0g3 = w0g.reshape(NF_G * 16, 2, HIDDEN0)
        w0s.append(w0g3[:, 0, :].astype(jnp.bfloat16))
        w0s.append(w0g3[:, 1, :].astype(jnp.bfloat16))
    return _mlp(embeds, dense_pad, w0s, w0d, b0, W1, b1, W2, b2)


# speedup vs baseline: 1.0173x; 1.0173x over previous
import functools

import jax
import jax.numpy as jnp
from jax import lax
from jax.experimental import pallas as pl
from jax.experimental.pallas import tpu as pltpu
from jax.experimental.pallas import tpu_sc as plsc

B = 16384
NUM_DENSE = 13
NF = 26
NG = 1
NF_G = NF // NG
VOCAB = 100000
EMB = 32
HIDDEN0 = 256

VCAP = 106496
TR_CHUNK = 8192

NC, NS = 2, 16
NW = NC * NS
SUBS = 8
CHUNK = SUBS * 128


def _tc_transpose(tables_t, fbase):
    q = TR_CHUNK // 8

    def body(*refs):
        out_ref = refs[-1]
        z = jnp.concatenate([r[0] for r in refs[:-1]], axis=0)
        zp = pltpu.bitcast(z.astype(jnp.bfloat16), jnp.float32)
        out_ref[...] = jnp.transpose(zp, (1, 0))

    last_blk = (VOCAB - 1) // q

    def in_spec(j):
        return pl.BlockSpec(
            (1, EMB, q),
            lambda f, c, j=j: (f + fbase, 0, jnp.minimum(8 * c + j, last_blk)))

    return pl.pallas_call(
        body,
        grid=(NF_G, VCAP // TR_CHUNK),
        in_specs=[in_spec(j) for j in range(8)],
        out_specs=pl.BlockSpec((q, 128),
                               lambda f, c: (f * (VCAP // TR_CHUNK) + c, 0)),
        out_shape=jax.ShapeDtypeStruct((NF_G * VCAP // 8, 128), jnp.float32),
    )(*([tables_t] * 8))


def _sc_gather(sparse2d, tables_flat, gname):
    rows = B * NF_G
    rw = rows // NW
    idx_rows = rw // 128
    nchunk = rw // CHUNK
    mesh = plsc.VectorSubcoreMesh(core_axis_name="c", subcore_axis_name="s")

    @functools.partial(
        pl.kernel,
        out_type=jax.ShapeDtypeStruct((rows, 16), jnp.float32),
        name=gname,
        mesh=mesh,
        compiler_params=pltpu.CompilerParams(use_tc_tiling_on_sc=False),
        scratch_types=[
            pltpu.VMEM((idx_rows, 128), jnp.int32),
            pltpu.VMEM((CHUNK, 16), jnp.float32),
            pltpu.SemaphoreType.DMA,
        ],
    )
    def k(sparse_hbm, tables_hbm, out_hbm, idx_v, rows_v, sem):
        wid = lax.axis_index("s") * NC + lax.axis_index("c")
        base = wid * rw

        pltpu.sync_copy(sparse_hbm.at[pl.ds(wid * idx_rows, idx_rows)], idx_v)

        def row_body(i, carry):
            for j in range(128 // 16):
                g0 = base + i * 128 + j * 16
                lanes = g0 + lax.iota(jnp.int32, 16)
                off = (lanes % NF_G) * VCAP
                sl = pl.ds(j * 16, 16)
                v = idx_v[i, sl]
                p = (((v >> 13) << 13) + ((v & 1023) << 3) + ((v >> 10) & 7))
                idx_v[i, sl] = off + p
            return carry

        lax.fori_loop(0, idx_rows, row_body, 0)

        def chunk_body(c, carry):
            handles = []
            for s in range(SUBS):
                h = pltpu.async_copy(
                    tables_hbm.at[idx_v.at[c * SUBS + s]],
                    rows_v.at[pl.ds(s * 128, 128)],
                    sem,
                )
                handles.append(h)
            for h in handles:
                h.wait()
            pltpu.sync_copy(rows_v, out_hbm.at[pl.ds(base + c * CHUNK, CHUNK)])
            return carry

        lax.fori_loop(0, nchunk, chunk_body, 0)

    return k(sparse2d, tables_flat)


def _mlp(embeds, dense_pad, w0s, w0d, b0, w1, b1, w2, b2, bs=1024):
    h0 = b0.shape[-1]
    h1 = b1.shape[-1]
    h2 = b2.shape[-1]
    ne = len(embeds)
    de = embeds[0].shape[-1]
    dd = dense_pad.shape[-1]

    def body(*refs):
        e_refs = refs[:ne]
        d_ref = refs[ne]
        w_refs = refs[ne + 1: ne + 1 + 2 * ne]
        w0d_ref, b0_ref, w1_ref, b1_ref, w2_ref, b2_ref, out_ref = \
            refs[ne + 1 + 2 * ne:]
        x = jnp.dot(d_ref[...], w0d_ref[...],
                    preferred_element_type=jnp.float32)
        for g, e_ref in enumerate(e_refs):
            u = pltpu.bitcast(e_ref[...], jnp.bfloat16)
            u3 = u.reshape(bs, 2, de)
            x = x + jnp.dot(u3[:, 0, :], w_refs[2 * g][...],
                            preferred_element_type=jnp.float32)
            x = x + jnp.dot(u3[:, 1, :], w_refs[2 * g + 1][...],
                            preferred_element_type=jnp.float32)
        x = jnp.maximum(x + b0_ref[...], 0.0)
        x = jnp.maximum(
            jnp.dot(x, w1_ref[...], preferred_element_type=jnp.float32)
            + b1_ref[...], 0.0)
        x = jnp.maximum(
            jnp.dot(x, w2_ref[...], preferred_element_type=jnp.float32)
            + b2_ref[...], 0.0)
        out_ref[...] = x

    def full(shape):
        return pl.BlockSpec(shape, lambda i: tuple(0 for _ in shape))

    return pl.pallas_call(
        body,
        grid=(B // bs,),
        in_specs=(
            [pl.BlockSpec((bs, de), lambda i: (i, 0)) for _ in range(ne)]
            + [pl.BlockSpec((bs, dd), lambda i: (i, 0))]
            + [full((de, h0)) for _ in range(2 * ne)]
            + [full((dd, h0)), full((1, h0)), full((h0, h1)), full((1, h1)),
               full((h1, h2)), full((1, h2))]
        ),
        out_specs=pl.BlockSpec((bs, h2), lambda i: (i, 0)),
        out_shape=jax.ShapeDtypeStruct((B, h2), jnp.float32),
    )(*embeds, dense_pad, *w0s, w0d, b0.reshape(1, h0), w1,
      b1.reshape(1, h1), w2, b2.reshape(1, h2))


def kernel(dense_inputs, sparse_inputs, tables, W0, b0, W1, b1, W2, b2):
    sparse_i = sparse_inputs.astype(jnp.int32)
    tables_t = jnp.transpose(tables, (0, 2, 1))

    embeds = []
    for g in range(NG):
        sp = sparse_i[:, g * NF_G:(g + 1) * NF_G].reshape(-1, 128)
        packed = _tc_transpose(tables_t, g * NF_G)
        raw = _sc_gather(sp, packed.reshape(NF_G * VCAP, 16),
                         f"sc_embed_gather{g}")
        embeds.append(raw.reshape(B, NF_G * 16))

    dense_pad = jnp.pad(dense_inputs, ((0, 0), (0, 128 - NUM_DENSE)))
    w0d = jnp.pad(W0[:NUM_DENSE], ((0, 128 - NUM_DENSE), (0, 0)))
    w0s = []
    for g in range(NG):
        w0g = W0[NUM_DENSE + g * NF_G * EMB: NUM_DENSE + (g + 1) * NF_G * EMB]
        w0g3 = w0g.reshape(NF_G * 16, 2, HIDDEN0)
        w0s.append(w0g3[:, 0, :].astype(jnp.bfloat16))
        w0s.append(w0g3[:, 1, :].astype(jnp.bfloat16))
    return _mlp(embeds, dense_pad, w0s, w0d, b0, W1, b1, W2, b2)


# speedup vs baseline: 1.0363x; 1.0186x over previous
import functools

import jax
import jax.numpy as jnp
from jax import lax
from jax.experimental import pallas as pl
from jax.experimental.pallas import tpu as pltpu
from jax.experimental.pallas import tpu_sc as plsc

B = 16384
NUM_DENSE = 13
NF = 26
NG = 1
NF_G = NF // NG
VOCAB = 100000
EMB = 32
HIDDEN0 = 256

VCAP = 106496
TR_CHUNK = 8192

NC, NS = 2, 16
NW = NC * NS
SUBS = 8
CHUNK = SUBS * 128


def _tc_transpose(tables_t, fbase):
    q = TR_CHUNK // 8

    def body(*refs):
        out_ref = refs[-1]
        z = jnp.concatenate([r[0] for r in refs[:-1]], axis=0)
        zp = pltpu.bitcast(z.astype(jnp.bfloat16), jnp.float32)
        out_ref[...] = jnp.transpose(zp, (1, 0))

    last_blk = (VOCAB - 1) // q

    def in_spec(j):
        return pl.BlockSpec(
            (1, EMB, q),
            lambda f, c, j=j: (f + fbase, 0, jnp.minimum(8 * c + j, last_blk)))

    return pl.pallas_call(
        body,
        grid=(NF_G, VCAP // TR_CHUNK),
        in_specs=[in_spec(j) for j in range(8)],
        out_specs=pl.BlockSpec((q, 128),
                               lambda f, c: (f * (VCAP // TR_CHUNK) + c, 0)),
        out_shape=jax.ShapeDtypeStruct((NF_G * VCAP // 8, 128), jnp.float32),
    )(*([tables_t] * 8))


def _sc_gather(sparse2d, tables_flat, gname):
    rows = B * NF_G
    rw = rows // NW
    idx_rows = rw // 128
    nchunk = rw // CHUNK
    mesh = plsc.VectorSubcoreMesh(core_axis_name="c", subcore_axis_name="s")

    @functools.partial(
        pl.kernel,
        out_type=jax.ShapeDtypeStruct((rows, 16), jnp.float32),
        name=gname,
        mesh=mesh,
        compiler_params=pltpu.CompilerParams(use_tc_tiling_on_sc=False),
        scratch_types=[
            pltpu.VMEM((idx_rows, 128), jnp.int32),
            pltpu.VMEM((CHUNK, 16), jnp.float32),
            pltpu.VMEM((CHUNK, 16), jnp.float32),
            pltpu.SemaphoreType.DMA,
            pltpu.SemaphoreType.DMA,
        ],
    )
    def k(sparse_hbm, tables_hbm, out_hbm, idx_v, rows_a, rows_b, sem_a,
          sem_b):
        wid = lax.axis_index("s") * NC + lax.axis_index("c")
        base = wid * rw

        pltpu.sync_copy(sparse_hbm.at[pl.ds(wid * idx_rows, idx_rows)], idx_v)

        def row_body(i, carry):
            for j in range(128 // 16):
                g0 = base + i * 128 + j * 16
                lanes = g0 + lax.iota(jnp.int32, 16)
                off = (lanes % NF_G) * VCAP
                sl = pl.ds(j * 16, 16)
                v = idx_v[i, sl]
                p = (((v >> 13) << 13) + ((v & 1023) << 3) + ((v >> 10) & 7))
                idx_v[i, sl] = off + p
            return carry

        lax.fori_loop(0, idx_rows, row_body, 0)

        def fire(c, buf, sem):
            for s in range(SUBS):
                pltpu.async_copy(
                    tables_hbm.at[idx_v.at[c * SUBS + s]],
                    buf.at[pl.ds(s * 128, 128)],
                    sem,
                )

        def drain(buf, sem):
            for s in range(SUBS):
                pltpu.make_async_copy(
                    tables_hbm.at[pl.ds(0, 128)],
                    buf.at[pl.ds(s * 128, 128)],
                    sem,
                ).wait()

        def write(c, buf):
            pltpu.sync_copy(buf, out_hbm.at[pl.ds(base + c * CHUNK, CHUNK)])

        fire(0, rows_a, sem_a)

        def pair_body(i, carry):
            c0 = 2 * i
            fire(c0 + 1, rows_b, sem_b)
            drain(rows_a, sem_a)
            write(c0, rows_a)
            fire(c0 + 2, rows_a, sem_a)
            drain(rows_b, sem_b)
            write(c0 + 1, rows_b)
            return carry

        lax.fori_loop(0, (nchunk - 1) // 2, pair_body, 0)
        drain(rows_a, sem_a)
        write(nchunk - 1, rows_a)

    return k(sparse2d, tables_flat)


def _mlp(embeds, dense_pad, w0s, w0d, b0, w1, b1, w2, b2, bs=1024):
    h0 = b0.shape[-1]
    h1 = b1.shape[-1]
    h2 = b2.shape[-1]
    ne = len(embeds)
    de = embeds[0].shape[-1]
    dd = dense_pad.shape[-1]

    def body(*refs):
        e_refs = refs[:ne]
        d_ref = refs[ne]
        w_refs = refs[ne + 1: ne + 1 + 2 * ne]
        w0d_ref, b0_ref, w1_ref, b1_ref, w2_ref, b2_ref, out_ref = \
            refs[ne + 1 + 2 * ne:]
        x = jnp.dot(d_ref[...], w0d_ref[...],
                    preferred_element_type=jnp.float32)
        for g, e_ref in enumerate(e_refs):
            u = pltpu.bitcast(e_ref[...], jnp.bfloat16)
            u3 = u.reshape(bs, 2, de)
            x = x + jnp.dot(u3[:, 0, :], w_refs[2 * g][...],
                            preferred_element_type=jnp.float32)
            x = x + jnp.dot(u3[:, 1, :], w_refs[2 * g + 1][...],
                            preferred_element_type=jnp.float32)
        x = jnp.maximum(x + b0_ref[...], 0.0)
        x = jnp.maximum(
            jnp.dot(x, w1_ref[...], preferred_element_type=jnp.float32)
            + b1_ref[...], 0.0)
        x = jnp.maximum(
            jnp.dot(x, w2_ref[...], preferred_element_type=jnp.float32)
            + b2_ref[...], 0.0)
        out_ref[...] = x

    def full(shape):
        return pl.BlockSpec(shape, lambda i: tuple(0 for _ in shape))

    return pl.pallas_call(
        body,
        grid=(B // bs,),
        in_specs=(
            [pl.BlockSpec((bs, de), lambda i: (i, 0)) for _ in range(ne)]
            + [pl.BlockSpec((bs, dd), lambda i: (i, 0))]
            + [full((de, h0)) for _ in range(2 * ne)]
            + [full((dd, h0)), full((1, h0)), full((h0, h1)), full((1, h1)),
               full((h1, h2)), full((1, h2))]
        ),
        out_specs=pl.BlockSpec((bs, h2), lambda i: (i, 0)),
        out_shape=jax.ShapeDtypeStruct((B, h2), jnp.float32),
    )(*embeds, dense_pad, *w0s, w0d, b0.reshape(1, h0), w1,
      b1.reshape(1, h1), w2, b2.reshape(1, h2))


def kernel(dense_inputs, sparse_inputs, tables, W0, b0, W1, b1, W2, b2):
    sparse_i = sparse_inputs.astype(jnp.int32)
    tables_t = jnp.transpose(tables, (0, 2, 1))

    embeds = []
    for g in range(NG):
        sp = sparse_i[:, g * NF_G:(g + 1) * NF_G].reshape(-1, 128)
        packed = _tc_transpose(tables_t, g * NF_G)
        raw = _sc_gather(sp, packed.reshape(NF_G * VCAP, 16),
                         f"sc_embed_gather{g}")
        embeds.append(raw.reshape(B, NF_G * 16))

    dense_pad = jnp.pad(dense_inputs, ((0, 0), (0, 128 - NUM_DENSE)))
    w0d = jnp.pad(W0[:NUM_DENSE], ((0, 128 - NUM_DENSE), (0, 0)))
    w0s = []
    for g in range(NG):
        w0g = W0[NUM_DENSE + g * NF_G * EMB: NUM_DENSE + (g + 1) * NF_G * EMB]
        w0g3 = w0g.reshape(NF_G * 16, 2, HIDDEN0)
        w0s.append(w0g3[:, 0, :].astype(jnp.bfloat16))
        w0s.append(w0g3[:, 1, :].astype(jnp.bfloat16))
    return _mlp(embeds, dense_pad, w0s, w0d, b0, W1, b1, W2, b2)
